# Initial kernel scaffold; baseline (speedup 1.0000x reference)
import jax
import jax.numpy as jnp
from jax.experimental import pallas as pl


def kernel(node_scalars, edge_index, edge_feats, W1, b1, W2, b2, gamma, beta):
    raise NotImplementedError("write your pallas kernel here")


# speedup vs baseline: 4.8583x; 4.8583x over previous
import jax
import jax.numpy as jnp
from jax import lax
from jax.experimental import pallas as pl
from jax.experimental.pallas import tpu as pltpu
from jax.experimental.pallas import tpu_sc as plsc

N_NODES = 10000
N_EDGES = 320000
D_NODE = 128
D_EDGE = 16

NC = 2
NS = 16
NW = NC * NS
EDGES_PER_BLK = 128
BPG = 8
GROUPS = 10
BLKS_PER_TILE = BPG * GROUPS
EDGES_PER_TILE = EDGES_PER_BLK * BLKS_PER_TILE
E_PAD = NW * EDGES_PER_TILE
PACK = 128 // D_EDGE
ROWS = N_EDGES // PACK


def _pq_body(ns_ref, wa_ref, wb_ref, p_ref, q_ref):
    ns = ns_ref[...]
    p_ref[...] = jnp.dot(ns, wa_ref[...], preferred_element_type=jnp.float32)
    q_ref[...] = jnp.dot(ns, wb_ref[...], preferred_element_type=jnp.float32)


def _precompute_pq(ns, wa, wb):
    br = 2000
    return pl.pallas_call(
        _pq_body,
        grid=(N_NODES // br,),
        in_specs=[
            pl.BlockSpec((br, D_NODE), lambda t: (t, 0)),
            pl.BlockSpec((D_NODE, D_EDGE), lambda t: (0, 0)),
            pl.BlockSpec((D_NODE, D_EDGE), lambda t: (0, 0)),
        ],
        out_specs=[
            pl.BlockSpec((br, D_EDGE), lambda t: (t, 0)),
            pl.BlockSpec((br, D_EDGE), lambda t: (t, 0)),
        ],
        out_shape=[jax.ShapeDtypeStruct((N_NODES, D_EDGE), jnp.float32)] * 2,
    )(ns, wa, wb)


def _gather_body(p_hbm, q_hbm, sidx_hbm, didx_hbm, gs_hbm, gd_hbm,
                 sidx_v, didx_v, gs_buf, gd_buf, gsem):
    wid = lax.axis_index("s") * NC + lax.axis_index("c")
    pltpu.sync_copy(sidx_hbm.at[wid], sidx_v)
    pltpu.sync_copy(didx_hbm.at[wid], didx_v)
    blk_base = wid * BLKS_PER_TILE

    @pl.loop(0, GROUPS)
    def _grp(g):
        cps = []
        for k in range(BPG):
            b = g * BPG + k
            cps.append(pltpu.async_copy(p_hbm.at[sidx_v.at[b]], gs_buf.at[k], gsem))
            cps.append(pltpu.async_copy(q_hbm.at[didx_v.at[b]], gd_buf.at[k], gsem))
        for cp in cps:
            cp.wait()
        off = blk_base + g * BPG
        pltpu.sync_copy(gs_buf, gs_hbm.at[pl.ds(off, BPG)])
        pltpu.sync_copy(gd_buf, gd_hbm.at[pl.ds(off, BPG)])


def _gather(p, q, sidx, didx):
    mesh = plsc.VectorSubcoreMesh(core_axis_name="c", subcore_axis_name="s")
    nblk = NW * BLKS_PER_TILE
    out = jax.ShapeDtypeStruct((nblk, EDGES_PER_BLK, D_EDGE), jnp.float32)
    f = pl.kernel(
        _gather_body,
        out_type=[out, out],
        mesh=mesh,
        scratch_types=[
            pltpu.VMEM((BLKS_PER_TILE, EDGES_PER_BLK), jnp.int32),
            pltpu.VMEM((BLKS_PER_TILE, EDGES_PER_BLK), jnp.int32),
            pltpu.VMEM((BPG, EDGES_PER_BLK, D_EDGE), jnp.float32),
            pltpu.VMEM((BPG, EDGES_PER_BLK, D_EDGE), jnp.float32),
            pltpu.SemaphoreType.DMA,
        ],
        compiler_params=pltpu.CompilerParams(use_tc_tiling_on_sc=False),
    )
    return f(p, q, sidx, didx)


def _dense_body(gs_ref, gd_ref, e_ref, w1_ref, w2_ref, ma_ref, pr_ref, o_ref):
    e = e_ref[...]
    x = (gs_ref[...] + gd_ref[...]
         + jnp.dot(e, w1_ref[...], preferred_element_type=jnp.float32)
         + pr_ref[0:1, :])
    h1 = x * (1.0 / (1.0 + jnp.exp(-x)))
    y = jnp.dot(h1, w2_ref[...], preferred_element_type=jnp.float32) + pr_ref[1:2, :]
    h2 = y * (1.0 / (1.0 + jnp.exp(-y)))
    z = e + h2
    m = jnp.dot(z, ma_ref[...], preferred_element_type=jnp.float32)
    s2 = jnp.dot(z * z, ma_ref[...], preferred_element_type=jnp.float32)
    var = s2 - m * m
    o_ref[...] = (z - m) * lax.rsqrt(var + 1e-5) * pr_ref[2:3, :] + pr_ref[3:4, :]


def _dense(gs_pk, gd_pk, e_pk, w1blk, w2blk, mavg, params):
    br = 2000
    full = lambda t: (0, 0)
    row = lambda t: (t, 0)
    return pl.pallas_call(
        _dense_body,
        grid=(ROWS // br,),
        in_specs=[
            pl.BlockSpec((br, 128), row),
            pl.BlockSpec((br, 128), row),
            pl.BlockSpec((br, 128), row),
            pl.BlockSpec((128, 128), full),
            pl.BlockSpec((128, 128), full),
            pl.BlockSpec((128, 128), full),
            pl.BlockSpec((8, 128), full),
        ],
        out_specs=pl.BlockSpec((br, 128), row),
        out_shape=jax.ShapeDtypeStruct((ROWS, 128), jnp.float32),
    )(gs_pk, gd_pk, e_pk, w1blk, w2blk, mavg, params)


def kernel(node_scalars, edge_index, edge_feats, W1, b1, W2, b2, gamma, beta):
    wa = W1[:D_NODE]
    wb = W1[D_NODE:2 * D_NODE]
    we = W1[2 * D_NODE:]

    p, q = _precompute_pq(node_scalars, wa, wb)

    pad = E_PAD - N_EDGES
    src = jnp.pad(edge_index[0].astype(jnp.int32), (0, pad))
    dst = jnp.pad(edge_index[1].astype(jnp.int32), (0, pad))
    sidx = src.reshape(NW, BLKS_PER_TILE, EDGES_PER_BLK)
    didx = dst.reshape(NW, BLKS_PER_TILE, EDGES_PER_BLK)

    gs, gd = _gather(p, q, sidx, didx)
    gs_pk = gs.reshape(-1, 128)
    gd_pk = gd.reshape(-1, 128)
    e_pk = edge_feats.reshape(ROWS, 128)

    eye = jnp.eye(PACK, dtype=jnp.float32)
    w1blk = jnp.kron(eye, we)
    w2blk = jnp.kron(eye, W2)
    mavg = jnp.kron(eye, jnp.full((D_EDGE, D_EDGE), 1.0 / D_EDGE, jnp.float32))
    params = jnp.concatenate([
        jnp.tile(b1, PACK)[None],
        jnp.tile(b2, PACK)[None],
        jnp.tile(gamma, PACK)[None],
        jnp.tile(beta, PACK)[None],
        jnp.zeros((4, 128), jnp.float32),
    ], axis=0)

    out_pk = _dense(gs_pk, gd_pk, e_pk, w1blk, w2blk, mavg, params)
    return out_pk.reshape(N_EDGES, D_EDGE)
